# baseline (speedup 1.0000x reference)
import functools

import jax
import jax.numpy as jnp
from jax import lax
from jax.experimental import pallas as pl
from jax.experimental.pallas import tpu as pltpu
from jax.experimental.pallas import tpu_sc as plsc

N = 10000
E = 320000
D = 128
HD = D // 2
CHUNK = 125
CPT = E // (16 * CHUNK)
ROWS_PT = N // 16
RCH = 125
NRC = 5
NBUF = 4


def _segsum_sc(srcp, dst2, xvb, xvf):
    mesh = plsc.VectorSubcoreMesh(core_axis_name="c", subcore_axis_name="s")

    @functools.partial(
        pl.kernel,
        out_type=jax.ShapeDtypeStruct((N, D), jnp.float32),
        mesh=mesh,
        compiler_params=pltpu.CompilerParams(use_tc_tiling_on_sc=False,
                                             needs_layout_passes=False),
        scratch_types=[
            pltpu.VMEM((CPT, CHUNK), jnp.int32),
            pltpu.VMEM((CPT, CHUNK), jnp.int32),
            *[pltpu.VMEM((CHUNK, HD), jnp.bfloat16) for _ in range(NBUF)],
            *[pltpu.VMEM((CHUNK, HD), jnp.float32) for _ in range(NBUF)],
            pltpu.VMEM_SHARED((N, HD), jnp.float32),
            *[pltpu.SemaphoreType.DMA for _ in range(2 * NBUF)],
        ],
    )
    def k(src_hbm, dst_hbm, xb_hbm, xf_hbm, out_hbm, src_v, dst_v, *rest):
        raw = list(rest[:NBUF])
        rows = list(rest[NBUF:2 * NBUF])
        agg_sh = rest[2 * NBUF]
        gsem = list(rest[2 * NBUF + 1:3 * NBUF + 1])
        ssem = list(rest[3 * NBUF + 1:])
        c = lax.axis_index("c")
        s = lax.axis_index("s")
        col0 = c * HD

        zero = jnp.zeros((16,), jnp.float32)

        def zrow(i, carry):
            for j in range(HD // 16):
                rows[0][i, pl.ds(j * 16, 16)] = zero
            return carry

        lax.fori_loop(0, RCH, zrow, 0)
        for t in range(NRC):
            pltpu.sync_copy(rows[0],
                            agg_sh.at[pl.ds(s * ROWS_PT + t * RCH, RCH)])
        plsc.subcore_barrier()

        row0 = s * CPT
        pltpu.sync_copy(src_hbm.at[c, pl.ds(row0, CPT)], src_v)
        pltpu.sync_copy(dst_hbm.at[pl.ds(row0, CPT)], dst_v)

        HB = NBUF // 2

        def draing(buf, sem):
            pltpu.make_async_copy(xb_hbm.at[pl.ds(0, CHUNK)], buf,
                                  sem).wait()

        def drainf(buf, sem):
            pltpu.make_async_copy(xf_hbm.at[pl.ds(0, CHUNK)], buf,
                                  sem).wait()

        def gather(j, b):
            pltpu.async_copy(xb_hbm.at[src_v.at[j]], raw[b], gsem[b])

        def scatter(j, b):
            pltpu.async_copy(rows[b], agg_sh.at[dst_v.at[j]], ssem[b],
                             add=True)

        def unpack(b):
            def conv_row(r, carry):
                for m in range(HD // 32):
                    lo, hi = plsc.unpack(raw[b][r, pl.ds(32 * m, 32)],
                                         format=plsc.PackFormat.INTERLEAVED)
                    rows[b][r, pl.ds(32 * m, 16)] = lo
                    rows[b][r, pl.ds(32 * m + 16, 16)] = hi
                return carry

            lax.fori_loop(0, CHUNK, conv_row, 0)

        for b in range(HB):
            gather(b, b)
        for b in range(NBUF):
            draing(raw[b], gsem[b])
            unpack(b)
            scatter(b, b)
            bn = (b + HB) % NBUF
            if b >= HB:
                drainf(rows[bn], ssem[bn])
            gather(b + HB, bn)

        def step(i, carry):
            for b in range(NBUF):
                j = i * NBUF + b
                bn = (b + HB) % NBUF
                draing(raw[b], gsem[b])
                unpack(b)
                scatter(j, b)
                drainf(rows[bn], ssem[bn])
                gather(jnp.minimum(j + HB, CPT - 1), bn)
            return carry

        lax.fori_loop(1, CPT // NBUF, step, 0)
        for b in range(NBUF):
            if b < HB:
                draing(raw[b], gsem[b])
            else:
                drainf(rows[b], ssem[b])
        plsc.subcore_barrier()

        for t in range(NRC):
            r0 = s * ROWS_PT + t * RCH
            b = t % 2
            if t >= 2:
                drainf(rows[b], gsem[b])
            pltpu.sync_copy(agg_sh.at[pl.ds(r0, RCH)], rows[b])
            pltpu.async_copy(rows[b],
                             out_hbm.at[pl.ds(r0, RCH), pl.ds(col0, HD)],
                             gsem[b])
        for b in range(2):
            drainf(rows[b], gsem[b])

    return k(srcp, dst2, xvb, xvf)


def _zpart(xin, wo, b):
    BR = 2000

    def body(x_ref, wo_ref, b_ref, o_ref):
        o_ref[...] = jnp.dot(x_ref[...], wo_ref[...],
                             preferred_element_type=jnp.float32) + b_ref[...]

    return pl.pallas_call(
        body,
        grid=(N // BR,),
        in_specs=[
            pl.BlockSpec((BR, D), lambda i: (i, 0)),
            pl.BlockSpec((D, D), lambda i: (0, 0)),
            pl.BlockSpec((1, D), lambda i: (0, 0)),
        ],
        out_specs=pl.BlockSpec((BR, D), lambda i: (i, 0)),
        out_shape=jax.ShapeDtypeStruct((N, D), jnp.float32),
    )(xin, wo, b)


def _combine(p, z, xin, wr, relu, residual, emit_bf16):
    BR = 2000

    def body(p_ref, z_ref, x_ref, wr_ref, o_ref, *obf):
        out = jnp.dot(p_ref[...], wr_ref[...],
                      preferred_element_type=jnp.float32) + z_ref[...]
        if relu:
            out = jnp.maximum(out, 0.0)
        if residual:
            out = out + x_ref[...]
        o_ref[...] = out
        if emit_bf16:
            obf[0][...] = out.astype(jnp.bfloat16)

    blk = pl.BlockSpec((BR, D), lambda i: (i, 0))
    out_specs = [blk, blk] if emit_bf16 else blk
    out_shape = ([jax.ShapeDtypeStruct((N, D), jnp.float32),
                  jax.ShapeDtypeStruct((N, D), jnp.bfloat16)]
                 if emit_bf16 else jax.ShapeDtypeStruct((N, D), jnp.float32))
    return pl.pallas_call(
        body,
        grid=(N // BR,),
        in_specs=[blk, blk, blk, pl.BlockSpec((D, D), lambda i: (0, 0))],
        out_specs=out_specs,
        out_shape=out_shape,
    )(p, z, xin, wr)


_DPERM = []
for _h in range(2):
    for _m in range(2):
        _base = 64 * _h + 32 * _m
        _DPERM += [_base + 2 * _k for _k in range(16)]
        _DPERM += [_base + 2 * _k + 1 for _k in range(16)]


def kernel(x, edge_index,
           W_rel1, W_root1, b1,
           W_rel2, W_root2, b2,
           W_rel3, W_root3, b3,
           W_rel4, W_root4, b4):
    ei = edge_index.astype(jnp.int32)
    src = ei[0].reshape(E // CHUNK, CHUNK)
    dst2 = ei[1].reshape(E // CHUNK, CHUNK)
    srcp = jnp.stack([2 * src, 2 * src + 1])
    dperm = jnp.asarray(_DPERM, jnp.int32)

    def layer(xin, xbf, wr, wo, b, relu, residual, emit_bf16):
        p = _segsum_sc(srcp, dst2, xbf.reshape(2 * N, HD),
                       xin.reshape(2 * N, HD))
        z = _zpart(xin, wo, b.reshape(1, D))
        return _combine(p, z, xin, wr[dperm, :], relu, residual, emit_bf16)

    xbf0 = x.astype(jnp.bfloat16)
    x1, xbf1 = layer(x, xbf0, W_rel1, W_root1, b1, True, False, True)
    x2, xbf2 = layer(x1, xbf1, W_rel2, W_root2, b2, True, True, True)
    x3, xbf3 = layer(x2, xbf2, W_rel3, W_root3, b3, False, True, True)
    x4 = layer(x3, xbf3, W_rel4, W_root4, b4, False, True, False)
    return x4


# speedup vs baseline: 1.7278x; 1.7278x over previous
import functools

import jax
import jax.numpy as jnp
from jax import lax
from jax.experimental import pallas as pl
from jax.experimental.pallas import tpu as pltpu
from jax.experimental.pallas import tpu_sc as plsc

N = 10000
E = 320000
D = 128
HD = D // 2
CHUNK = 125
CPT = E // (16 * CHUNK)
ROWS_PT = N // 16
RCH = 125
NRC = 5
NBUF = 4


def _segsum_sc(srcp, dst2, xv):
    mesh = plsc.VectorSubcoreMesh(core_axis_name="c", subcore_axis_name="s")

    @functools.partial(
        pl.kernel,
        out_type=jax.ShapeDtypeStruct((N, D), jnp.float32),
        mesh=mesh,
        compiler_params=pltpu.CompilerParams(use_tc_tiling_on_sc=False),
        scratch_types=[
            pltpu.VMEM((CPT, CHUNK), jnp.int32),
            pltpu.VMEM((CPT, CHUNK), jnp.int32),
            *[pltpu.VMEM((CHUNK, HD), jnp.float32) for _ in range(NBUF)],
            pltpu.VMEM_SHARED((N, HD), jnp.float32),
            *[pltpu.SemaphoreType.DMA for _ in range(2 * NBUF)],
        ],
    )
    def k(src_hbm, dst_hbm, x_hbm, out_hbm, src_v, dst_v, *rest):
        rows = list(rest[:NBUF])
        agg_sh = rest[NBUF]
        gsem = list(rest[NBUF + 1:2 * NBUF + 1])
        ssem = list(rest[2 * NBUF + 1:])
        c = lax.axis_index("c")
        s = lax.axis_index("s")
        col0 = c * HD

        zero = jnp.zeros((16,), jnp.float32)

        def zrow(i, carry):
            for j in range(HD // 16):
                rows[0][i, pl.ds(j * 16, 16)] = zero
            return carry

        lax.fori_loop(0, RCH, zrow, 0)
        for t in range(NRC):
            pltpu.sync_copy(rows[0],
                            agg_sh.at[pl.ds(s * ROWS_PT + t * RCH, RCH)])
        plsc.subcore_barrier()

        row0 = s * CPT
        pltpu.sync_copy(src_hbm.at[c, pl.ds(row0, CPT)], src_v)
        pltpu.sync_copy(dst_hbm.at[pl.ds(row0, CPT)], dst_v)

        HB = NBUF // 2

        def drain(buf, sem):
            pltpu.make_async_copy(x_hbm.at[pl.ds(0, CHUNK)], buf, sem).wait()

        def gather(j, b):
            pltpu.async_copy(x_hbm.at[src_v.at[j]], rows[b], gsem[b])

        def scatter(j, b):
            pltpu.async_copy(rows[b], agg_sh.at[dst_v.at[j]], ssem[b],
                             add=True)

        for b in range(HB):
            gather(b, b)
        for b in range(NBUF):
            drain(rows[b], gsem[b])
            scatter(b, b)
            bn = (b + HB) % NBUF
            if b >= HB:
                drain(rows[bn], ssem[bn])
            gather(b + HB, bn)

        def step(i, carry):
            for b in range(NBUF):
                j = i * NBUF + b
                bn = (b + HB) % NBUF
                drain(rows[b], gsem[b])
                scatter(j, b)
                drain(rows[bn], ssem[bn])
                gather(jnp.minimum(j + HB, CPT - 1), bn)
            return carry

        lax.fori_loop(1, CPT // NBUF, step, 0)
        for b in range(NBUF):
            if b < HB:
                drain(rows[b], gsem[b])
            else:
                drain(rows[b], ssem[b])
        plsc.subcore_barrier()

        for t in range(NRC):
            r0 = s * ROWS_PT + t * RCH
            b = t % 2
            if t >= 2:
                drain(rows[b], gsem[b])
            pltpu.sync_copy(agg_sh.at[pl.ds(r0, RCH)], rows[b])
            pltpu.async_copy(rows[b],
                             out_hbm.at[pl.ds(r0, RCH), pl.ds(col0, HD)],
                             gsem[b])
        for b in range(2):
            drain(rows[b], gsem[b])

    return k(srcp, dst2, xv)


def _zpart(xin, wo, b):
    BR = 2000

    def body(x_ref, wo_ref, b_ref, o_ref):
        o_ref[...] = jnp.dot(x_ref[...], wo_ref[...],
                             preferred_element_type=jnp.float32) + b_ref[...]

    return pl.pallas_call(
        body,
        grid=(N // BR,),
        in_specs=[
            pl.BlockSpec((BR, D), lambda i: (i, 0)),
            pl.BlockSpec((D, D), lambda i: (0, 0)),
            pl.BlockSpec((1, D), lambda i: (0, 0)),
        ],
        out_specs=pl.BlockSpec((BR, D), lambda i: (i, 0)),
        out_shape=jax.ShapeDtypeStruct((N, D), jnp.float32),
    )(xin, wo, b)


def _combine(p, z, xin, wr, relu, residual):
    BR = 2000

    def body(p_ref, z_ref, x_ref, wr_ref, o_ref):
        out = jnp.dot(p_ref[...], wr_ref[...],
                      preferred_element_type=jnp.float32) + z_ref[...]
        if relu:
            out = jnp.maximum(out, 0.0)
        if residual:
            out = out + x_ref[...]
        o_ref[...] = out

    blk = pl.BlockSpec((BR, D), lambda i: (i, 0))
    return pl.pallas_call(
        body,
        grid=(N // BR,),
        in_specs=[blk, blk, blk, pl.BlockSpec((D, D), lambda i: (0, 0))],
        out_specs=blk,
        out_shape=jax.ShapeDtypeStruct((N, D), jnp.float32),
    )(p, z, xin, wr)


def kernel(x, edge_index,
           W_rel1, W_root1, b1,
           W_rel2, W_root2, b2,
           W_rel3, W_root3, b3,
           W_rel4, W_root4, b4):
    ei = edge_index.astype(jnp.int32)
    src = ei[0].reshape(E // CHUNK, CHUNK)
    dst2 = ei[1].reshape(E // CHUNK, CHUNK)
    srcp = jnp.stack([2 * src, 2 * src + 1])

    def layer(xin, wr, wo, b, relu, residual):
        p = _segsum_sc(srcp, dst2, xin.reshape(2 * N, HD))
        z = _zpart(xin, wo, b.reshape(1, D))
        return _combine(p, z, xin, wr, relu, residual)

    x1 = layer(x, W_rel1, W_root1, b1, True, False)
    x2 = layer(x1, W_rel2, W_root2, b2, True, True)
    x3 = layer(x2, W_rel3, W_root3, b3, False, True)
    x4 = layer(x3, W_rel4, W_root4, b4, False, True)
    return x4
